# Initial kernel scaffold; baseline (speedup 1.0000x reference)
import jax
import jax.numpy as jnp
from jax.experimental import pallas as pl


def kernel(hidden_states, Wr, w1, w2):
    raise NotImplementedError("write your pallas kernel here")


# speedup vs baseline: 1.2961x; 1.2961x over previous
import jax
import jax.numpy as jnp
from jax.experimental import pallas as pl
from jax.experimental.pallas import tpu as pltpu

HD, ID, NE, TOPK = 768, 3072, 8, 2
TOK = 2048
AUX = 0.001
IC = 1536
NIC = ID // IC


def _router_kernel(x_ref, wr_ref, wcomb_ref, lb_ref, z_ref):
    x = x_ref[...]
    logits = jnp.dot(x, wr_ref[...], preferred_element_type=jnp.float32)
    m = jnp.max(logits, axis=-1, keepdims=True)
    ex = jnp.exp(logits - m)
    se = jnp.sum(ex, axis=-1, keepdims=True)
    probs = ex / se
    eidx = jax.lax.broadcasted_iota(jnp.int32, probs.shape, 1)
    v1 = jnp.max(probs, axis=-1, keepdims=True)
    i1 = jnp.min(jnp.where(probs == v1, eidx, NE), axis=-1, keepdims=True)
    oh1 = eidx == i1
    probs_m = jnp.where(oh1, -1.0, probs)
    v2 = jnp.max(probs_m, axis=-1, keepdims=True)
    i2 = jnp.min(jnp.where(probs_m == v2, eidx, NE), axis=-1, keepdims=True)
    oh2 = eidx == i2
    denom = v1 + v2
    wcomb = (jnp.where(oh1, v1, 0.0) + jnp.where(oh2, v2, 0.0)) / denom
    wcomb_ref[...] = wcomb
    counts = jnp.sum(oh1.astype(jnp.float32) + oh2.astype(jnp.float32), axis=0)
    frac_tokens = counts / (TOK * TOPK)
    mean_probs = jnp.mean(probs, axis=0)
    lb_ref[...] = (AUX * NE * jnp.sum(frac_tokens * mean_probs)).reshape(1, 1)
    lse = m + jnp.log(se)
    z_ref[...] = jnp.mean(lse * lse).reshape(1, 1)


def _expert_kernel(wcomb_ref, x_ref, w1_ref, w2_ref, out_ref):
    e = pl.program_id(0)
    ic = pl.program_id(1)

    @pl.when((e == 0) & (ic == 0))
    def _():
        out_ref[...] = jnp.zeros_like(out_ref)

    x = x_ref[...]
    h = jnp.dot(x, w1_ref[0], preferred_element_type=jnp.float32)
    h = jax.nn.gelu(h)
    part = jnp.dot(h, w2_ref[0], preferred_element_type=jnp.float32)
    wcomb = wcomb_ref[...]
    lane = jax.lax.broadcasted_iota(jnp.int32, wcomb.shape, 1)
    wc = jnp.sum(jnp.where(lane == e, wcomb, 0.0), axis=1, keepdims=True)
    out_ref[...] += wc * part


def kernel(hidden_states, Wr, w1, w2):
    b, s, h = hidden_states.shape
    x = hidden_states.reshape(-1, h).astype(jnp.float32)

    wcomb, lb, z = pl.pallas_call(
        _router_kernel,
        out_shape=[
            jax.ShapeDtypeStruct((TOK, NE), jnp.float32),
            jax.ShapeDtypeStruct((1, 1), jnp.float32),
            jax.ShapeDtypeStruct((1, 1), jnp.float32),
        ],
    )(x, Wr)

    out = pl.pallas_call(
        _expert_kernel,
        grid=(NE, NIC),
        in_specs=[
            pl.BlockSpec((TOK, NE), lambda e, ic: (0, 0)),
            pl.BlockSpec((TOK, HD), lambda e, ic: (0, 0)),
            pl.BlockSpec((1, HD, IC), lambda e, ic: (e, 0, ic)),
            pl.BlockSpec((1, IC, HD), lambda e, ic: (e, ic, 0)),
        ],
        out_specs=pl.BlockSpec((TOK, HD), lambda e, ic: (0, 0)),
        out_shape=jax.ShapeDtypeStruct((TOK, HD), jnp.float32),
    )(wcomb, x, w1, w2)

    return out.reshape(b, s, h), lb[0, 0], z[0, 0]
